# Initial kernel scaffold; baseline (speedup 1.0000x reference)
import jax
import jax.numpy as jnp
from jax.experimental import pallas as pl


def kernel(x, weights):
    raise NotImplementedError("write your pallas kernel here")


# speedup vs baseline: 1.0167x; 1.0167x over previous
import jax
import jax.numpy as jnp
from jax import lax
from jax.experimental import pallas as pl
from jax.experimental.pallas import tpu as pltpu
from jax.experimental.pallas import tpu_sc as plsc

_NC = 2
_NS = 16
_NW = _NC * _NS

_B = 16384 * 200
_D = 16
_CHUNK = 2048
_NBUF = 2


def _gather_body(table_hbm, idx_hbm, out_hbm,
                 idx0, idx1, rows0, rows1, sem0, sem1):
    idx_bufs = (idx0, idx1)
    row_bufs = (rows0, rows1)
    sems = (sem0, sem1)
    wid = lax.axis_index("s") * _NC + lax.axis_index("c")
    b_per_w = _B // _NW
    nchunks = b_per_w // _CHUNK
    base = wid * b_per_w

    @pl.loop(0, nchunks, step=_NBUF)
    def _group(g0):
        descs = []
        for b in range(_NBUF):
            off = base + (g0 + b) * _CHUNK
            pltpu.sync_copy(idx_hbm.at[pl.ds(off, _CHUNK)], idx_bufs[b])
            descs.append(
                pltpu.async_copy(table_hbm.at[idx_bufs[b]], row_bufs[b], sems[b]))
        for b in range(_NBUF):
            off = base + (g0 + b) * _CHUNK
            descs[b].wait()
            pltpu.sync_copy(row_bufs[b], out_hbm.at[pl.ds(off, _CHUNK)])


@jax.jit
def _gather(weights, idx):
    mesh = plsc.VectorSubcoreMesh(core_axis_name="c", subcore_axis_name="s")
    f = pl.kernel(
        _gather_body,
        out_type=jax.ShapeDtypeStruct((_B, _D), jnp.float32),
        mesh=mesh,
        scratch_types=[
            pltpu.VMEM((_CHUNK,), jnp.int32),
            pltpu.VMEM((_CHUNK,), jnp.int32),
            pltpu.VMEM((_CHUNK, _D), jnp.float32),
            pltpu.VMEM((_CHUNK, _D), jnp.float32),
            pltpu.SemaphoreType.DMA,
            pltpu.SemaphoreType.DMA,
        ],
        compiler_params=pltpu.CompilerParams(use_tc_tiling_on_sc=False),
    )
    return f(weights, idx)


def kernel(x, weights):
    idx = x[:, 2:].astype(jnp.int32).reshape(-1)
    mean = _gather(weights, idx)
    sigma = jnp.array([1.0], dtype=jnp.float32)
    return (mean, sigma)


# speedup vs baseline: 1.4744x; 1.4502x over previous
import jax
import jax.numpy as jnp
from jax import lax
from jax.experimental import pallas as pl
from jax.experimental.pallas import tpu as pltpu
from jax.experimental.pallas import tpu_sc as plsc

_NC = 2
_NS = 16
_NW = _NC * _NS

_V = 1_000_000
_B = 16384 * 200
_D = 16
_CHUNK = 1024
_NBUF = 2
_TPC = _CHUNK // 128 * 8


def _gather_body(table_hbm, idx_hbm, out_hbm,
                 idx0, idx1, rows0, rows1, t0, t1, sem0, sem1):
    idx_bufs = (idx0, idx1)
    row_bufs = (rows0, rows1)
    t_bufs = (t0, t1)
    sems = (sem0, sem1)
    wid = lax.axis_index("s") * _NC + lax.axis_index("c")
    b_per_w = _B // _NW
    nchunks = b_per_w // _CHUNK
    base = wid * b_per_w

    iota16 = lax.iota(jnp.int32, 16)
    col_consts = [jnp.full((16,), j, jnp.int32) for j in range(_D)]

    @pl.loop(0, nchunks, step=_NBUF)
    def _group(g0):
        descs = []
        for b in range(_NBUF):
            off = base + (g0 + b) * _CHUNK
            pltpu.sync_copy(idx_hbm.at[pl.ds(off, _CHUNK)], idx_bufs[b])
            descs.append(
                pltpu.async_copy(table_hbm.at[idx_bufs[b]], row_bufs[b], sems[b]))
        for b in range(_NBUF):
            off = base + (g0 + b) * _CHUNK
            descs[b].wait()
            rows = row_bufs[b]
            tbuf = t_bufs[b]

            @pl.loop(0, _CHUNK // 128)
            def _bt(bt):
                rowbase = bt * 128
                for c0 in range(8):
                    rowvec = iota16 + (rowbase + c0 * 16)
                    for j in range(_D):
                        jt, r = divmod(j, 8)
                        v = plsc.load_gather(rows, [rowvec, col_consts[j]])
                        tbuf[jt * _TPC + bt * 8 + r, pl.ds(c0 * 16, 16)] = v

            q0 = off // 128 * 8
            pltpu.sync_copy(tbuf.at[pl.ds(0, _TPC), :],
                            out_hbm.at[pl.ds(q0, _TPC), :])
            pltpu.sync_copy(tbuf.at[pl.ds(_TPC, _TPC), :],
                            out_hbm.at[pl.ds(_B // 128 * 8 + q0, _TPC), :])


@jax.jit
def _gather(weights2d, idx):
    mesh = plsc.VectorSubcoreMesh(core_axis_name="c", subcore_axis_name="s")
    f = pl.kernel(
        _gather_body,
        out_type=jax.ShapeDtypeStruct((_B * _D // 128, 128), jnp.float32),
        mesh=mesh,
        scratch_types=[
            pltpu.VMEM((_CHUNK,), jnp.int32),
            pltpu.VMEM((_CHUNK,), jnp.int32),
            pltpu.VMEM((_CHUNK, _D), jnp.float32),
            pltpu.VMEM((_CHUNK, _D), jnp.float32),
            pltpu.VMEM((2 * _TPC, 128), jnp.float32),
            pltpu.VMEM((2 * _TPC, 128), jnp.float32),
            pltpu.SemaphoreType.DMA,
            pltpu.SemaphoreType.DMA,
        ],
        compiler_params=pltpu.CompilerParams(
            use_tc_tiling_on_sc=False, needs_layout_passes=False),
    )
    return f(weights2d, idx)


def kernel(x, weights):
    idx = x[:, 2:].astype(jnp.int32).reshape(-1)
    out = _gather(weights, idx)
    mean = (out.reshape(2, _B // 128, 8, 128)
            .transpose(1, 3, 0, 2)
            .reshape(_B, _D))
    sigma = jnp.array([1.0], dtype=jnp.float32)
    return (mean, sigma)
